# baseline (speedup 1.0000x reference)
import numpy as np
import jax
import jax.numpy as jnp
from jax.experimental import pallas as pl
from jax.experimental.pallas import tpu as pltpu

D = 1024; H = 16; KV = 4; HD = D // H; FF = 512; E = 64; TOPK = 2
B = 32; S = 8; T = B * S; A = T * TOPK
EPS = 1e-6
ROPE_THETA = 1000000.0
TM = 64
NT = A // TM
NV = NT + E - 1


def _rope_const():
    inv_freq = (1.0 / (ROPE_THETA ** (np.arange(0, HD, 2, dtype=np.float32) / HD)))
    pos = np.arange(S, dtype=np.float32)
    freqs = np.outer(pos, inv_freq)
    emb = np.concatenate([freqs, freqs], axis=-1)
    cos = np.tile(np.cos(emb).astype(np.float32), (B, 1))
    sin = np.tile(np.sin(emb).astype(np.float32), (B, 1))
    return jnp.asarray(cos), jnp.asarray(sin)


def _rms(x, w):
    return x * jax.lax.rsqrt(jnp.mean(x * x, axis=-1, keepdims=True) + EPS) * w


def _rope(x, cos, sin):
    half = HD // 2
    rot = jnp.concatenate([-x[:, half:], x[:, :half]], axis=1)
    return x * cos + rot * sin


def _attn_kernel(x_ref, ln1_ref, wq_ref, wk_ref, wv_ref, wo_ref, qn_ref,
                 kn_ref, cos_ref, sin_ref, o_ref):
    x = x_ref[...]
    h = _rms(x, ln1_ref[...])
    q = jnp.dot(h, wq_ref[...], preferred_element_type=jnp.float32)
    k = jnp.dot(h, wk_ref[...], preferred_element_type=jnp.float32)
    v = jnp.dot(h, wv_ref[...], preferred_element_type=jnp.float32)
    cos = cos_ref[...]
    sin = sin_ref[...]
    r = jax.lax.broadcasted_iota(jnp.int32, (T, T), 0)
    c = jax.lax.broadcasted_iota(jnp.int32, (T, T), 1)
    valid = (r // S == c // S) & (c <= r)
    khs, vhs = [], []
    for j in range(KV):
        kh = _rope(_rms(k[:, j * HD:(j + 1) * HD], kn_ref[...]), cos, sin)
        khs.append(kh)
        vhs.append(v[:, j * HD:(j + 1) * HD])
    scale = 1.0 / np.sqrt(HD).astype(np.float32)
    ctxs = []
    for i in range(H):
        qh = _rope(_rms(q[:, i * HD:(i + 1) * HD], qn_ref[...]), cos, sin)
        kh = khs[i // (H // KV)]
        vh = vhs[i // (H // KV)]
        s_ = jax.lax.dot_general(qh, kh, (((1,), (1,)), ((), ())),
                                 preferred_element_type=jnp.float32) * scale
        s_ = jnp.where(valid, s_, -1e9)
        m = jnp.max(s_, axis=1, keepdims=True)
        ex = jnp.exp(s_ - m)
        p_ = ex / jnp.sum(ex, axis=1, keepdims=True)
        ctxs.append(jnp.dot(p_, vh, preferred_element_type=jnp.float32))
    ctx = jnp.concatenate(ctxs, axis=1)
    o_ref[...] = x + jnp.dot(ctx, wo_ref[...], preferred_element_type=jnp.float32)


def _attn_route_kernel(x_ref, ln1_ref, wq_ref, wk_ref, wv_ref, wo_ref, qn_ref,
                       kn_ref, cos_ref, sin_ref, ln2_ref, wr_ref,
                       xa_ref, xs_ref, sg_ref, eid_ref, tid_ref, offs_ref,
                       cnts_ref, aux_ref):
    _attn_kernel(x_ref, ln1_ref, wq_ref, wk_ref, wv_ref, wo_ref, qn_ref,
                 kn_ref, cos_ref, sin_ref, xa_ref)
    xa = xa_ref[...]
    h2 = _rms(xa, ln2_ref[...])
    logits = jnp.dot(h2, wr_ref[...], preferred_element_type=jnp.float32)
    m = jnp.max(logits, axis=1, keepdims=True)
    ex = jnp.exp(logits - m)
    probs = ex / jnp.sum(ex, axis=1, keepdims=True)
    lane = jax.lax.broadcasted_iota(jnp.int32, (T, E), 1)
    v1 = jnp.max(probs, axis=1, keepdims=True)
    i1 = jnp.min(jnp.where(probs == v1, lane, E), axis=1, keepdims=True)
    oh1 = lane == i1
    probs_m = jnp.where(oh1, -1.0, probs)
    v2 = jnp.max(probs_m, axis=1, keepdims=True)
    i2 = jnp.min(jnp.where(probs_m == v2, lane, E), axis=1, keepdims=True)
    oh2 = lane == i2
    sg = v1 + v2
    g1 = v1 / sg
    g2 = v2 / sg
    ohall = oh1.astype(jnp.float32) + oh2.astype(jnp.float32)
    f_ = jnp.sum(ohall, axis=0, keepdims=True) / (T * TOPK)
    p_ = jnp.sum(probs, axis=0, keepdims=True) / T
    aux_ref[...] = E * jnp.sum(f_ * p_, axis=1, keepdims=True)
    rT = jax.lax.broadcasted_iota(jnp.int32, (T, T), 0)
    cT = jax.lax.broadcasted_iota(jnp.int32, (T, T), 1)
    ltok = (cT < rT).astype(jnp.float32)
    rank = jax.lax.dot_general(ltok, ohall, (((1,), (0,)), ((), ())),
                               preferred_element_type=jnp.float32)
    rE = jax.lax.broadcasted_iota(jnp.int32, (E, E), 0)
    cE = jax.lax.broadcasted_iota(jnp.int32, (E, E), 1)
    lexp_row = (rE < cE).astype(jnp.float32)
    lexp_col = (cE < rE).astype(jnp.float32)
    hist_row = jnp.sum(ohall, axis=0, keepdims=True)
    off_row = jnp.dot(hist_row, lexp_row, preferred_element_type=jnp.float32)
    pos_te = off_row + rank
    pos1 = jnp.sum(jnp.where(oh1, pos_te, 0.0), axis=1, keepdims=True)
    pos2 = jnp.sum(jnp.where(oh2, pos_te, 0.0), axis=1, keepdims=True)
    pcol = jax.lax.broadcasted_iota(jnp.int32, (T, A), 1)
    m1 = (pcol == pos1.astype(jnp.int32)).astype(jnp.float32)
    m2 = (pcol == pos2.astype(jnp.int32)).astype(jnp.float32)
    smat = m1 + m2
    xs_ref[...] = jax.lax.dot_general(smat, h2, (((0,), (0,)), ((), ())),
                                      preferred_element_type=jnp.float32)
    sg_ref[...] = m1 * g1 + m2 * g2
    ones_t = jnp.ones((T, 1), jnp.float32)
    histc = jax.lax.dot_general(ohall, ones_t, (((0,), (0,)), ((), ())),
                                preferred_element_type=jnp.float32)
    offc = jnp.dot(lexp_col, histc, preferred_element_type=jnp.float32)
    tile0 = jnp.floor(offc / TM)
    tile1 = jnp.floor((offc + histc - 1.0) / TM)
    ntiles = jnp.where(histc > 0, tile1 - tile0 + 1.0, 0.0)
    cumv = jnp.dot(lexp_col, ntiles, preferred_element_type=jnp.float32)
    vcol = jax.lax.broadcasted_iota(jnp.int32, (E, NV), 1).astype(jnp.float32)
    erow = jax.lax.broadcasted_iota(jnp.int32, (E, NV), 0).astype(jnp.float32)
    active = (vcol >= cumv) & (vcol < cumv + ntiles)
    activef = active.astype(jnp.float32)
    validv = jnp.sum(activef, axis=0, keepdims=True)
    eidv = jnp.sum(jnp.where(active, erow, 0.0), axis=0, keepdims=True)
    tidv = jnp.sum(jnp.where(active, tile0 + (vcol - cumv), 0.0),
                   axis=0, keepdims=True)
    eid_ref[...] = jnp.where(validv > 0, eidv, float(E - 1)).astype(jnp.int32)
    tid_ref[...] = jnp.where(validv > 0, tidv, float(NT - 1)).astype(jnp.int32)
    offs_ref[...] = offc.astype(jnp.int32).reshape(1, E)
    cnts_ref[...] = histc.astype(jnp.int32).reshape(1, E)


def _moe_kernel(eid_ref, tid_ref, offs_ref, cnts_ref,
                x_ref, wg_ref, wu_ref, wd_ref, y_ref):
    i = pl.program_id(0)
    e = eid_ref[0, i]
    t = tid_ref[0, i]
    off = offs_ref[0, e]
    cnt = cnts_ref[0, e]
    rs = jnp.maximum(off - t * TM, 0)
    re = jnp.minimum(off + cnt - t * TM, TM)

    @pl.when(re > rs)
    def _():
        xt = x_ref[...]
        g = jnp.dot(xt, wg_ref[0], preferred_element_type=jnp.float32)
        u = jnp.dot(xt, wu_ref[0], preferred_element_type=jnp.float32)
        hid = (g / (1.0 + jnp.exp(-g))) * u
        y = jnp.dot(hid, wd_ref[0], preferred_element_type=jnp.float32)
        rows = jax.lax.broadcasted_iota(jnp.int32, (TM, D), 0)
        mask = (rows >= rs) & (rows < re)
        y_ref[...] = jnp.where(mask, y, y_ref[...])


def _comb_kernel(xa_ref, sg_ref, y_ref, o_ref):
    o_ref[...] = xa_ref[...] + jnp.dot(sg_ref[...], y_ref[...],
                                       preferred_element_type=jnp.float32)


def kernel(x, ln1_w, wq, wk, wv, wo, q_norm_w, k_norm_w, ln2_w, w_router,
           wg, wu, wd):
    x2 = x.reshape(T, D)
    cos, sin = _rope_const()
    xa, xs, sgm, eid, tid, offs, cnts, aux = pl.pallas_call(
        _attn_route_kernel,
        out_shape=(
            jax.ShapeDtypeStruct((T, D), jnp.float32),
            jax.ShapeDtypeStruct((A, D), jnp.float32),
            jax.ShapeDtypeStruct((T, A), jnp.float32),
            jax.ShapeDtypeStruct((1, NV), jnp.int32),
            jax.ShapeDtypeStruct((1, NV), jnp.int32),
            jax.ShapeDtypeStruct((1, E), jnp.int32),
            jax.ShapeDtypeStruct((1, E), jnp.int32),
            jax.ShapeDtypeStruct((1, 1), jnp.float32),
        ),
    )(x2, ln1_w.reshape(1, D), wq, wk, wv, wo,
      q_norm_w.reshape(1, HD), k_norm_w.reshape(1, HD), cos, sin,
      ln2_w.reshape(1, D), w_router)

    grid_spec = pltpu.PrefetchScalarGridSpec(
        num_scalar_prefetch=4,
        grid=(NV,),
        in_specs=[
            pl.BlockSpec((TM, D), lambda i, eid, tid, offs, cnts: (tid[0, i], 0)),
            pl.BlockSpec((1, D, FF), lambda i, eid, tid, offs, cnts: (eid[0, i], 0, 0)),
            pl.BlockSpec((1, D, FF), lambda i, eid, tid, offs, cnts: (eid[0, i], 0, 0)),
            pl.BlockSpec((1, FF, D), lambda i, eid, tid, offs, cnts: (eid[0, i], 0, 0)),
        ],
        out_specs=pl.BlockSpec((TM, D), lambda i, eid, tid, offs, cnts: (tid[0, i], 0)),
    )
    y = pl.pallas_call(
        _moe_kernel,
        grid_spec=grid_spec,
        out_shape=jax.ShapeDtypeStruct((A, D), jnp.float32),
    )(eid, tid, offs, cnts, xs, wg, wu, wd)

    out = pl.pallas_call(
        _comb_kernel,
        out_shape=jax.ShapeDtypeStruct((T, D), jnp.float32),
    )(xa, sgm, y)
    return out.reshape(B, S, D), aux[0, 0]


# speedup vs baseline: 1.0393x; 1.0393x over previous
import numpy as np
import jax
import jax.numpy as jnp
from jax.experimental import pallas as pl
from jax.experimental.pallas import tpu as pltpu

D = 1024; H = 16; KV = 4; HD = D // H; FF = 512; E = 64; TOPK = 2
B = 32; S = 8; T = B * S; A = T * TOPK
EPS = 1e-6
ROPE_THETA = 1000000.0
TM = 64
P = 1024


def _rope_const():
    inv_freq = (1.0 / (ROPE_THETA ** (np.arange(0, HD, 2, dtype=np.float32) / HD)))
    pos = np.arange(S, dtype=np.float32)
    freqs = np.outer(pos, inv_freq)
    emb = np.concatenate([freqs, freqs], axis=-1)
    cos = np.tile(np.cos(emb).astype(np.float32), (B, 1))
    sin = np.tile(np.sin(emb).astype(np.float32), (B, 1))
    return jnp.asarray(cos), jnp.asarray(sin)


def _rms(x, w):
    return x * jax.lax.rsqrt(jnp.mean(x * x, axis=-1, keepdims=True) + EPS) * w


def _rope(x, cos, sin):
    half = HD // 2
    rot = jnp.concatenate([-x[:, half:], x[:, :half]], axis=1)
    return x * cos + rot * sin


def _attn_kernel(x_ref, ln1_ref, wq_ref, wk_ref, wv_ref, wo_ref, qn_ref,
                 kn_ref, cos_ref, sin_ref, o_ref):
    x = x_ref[...]
    h = _rms(x, ln1_ref[...])
    q = jnp.dot(h, wq_ref[...], preferred_element_type=jnp.float32)
    k = jnp.dot(h, wk_ref[...], preferred_element_type=jnp.float32)
    v = jnp.dot(h, wv_ref[...], preferred_element_type=jnp.float32)
    cos = cos_ref[...]
    sin = sin_ref[...]
    r = jax.lax.broadcasted_iota(jnp.int32, (T, T), 0)
    c = jax.lax.broadcasted_iota(jnp.int32, (T, T), 1)
    valid = (r // S == c // S) & (c <= r)
    khs, vhs = [], []
    for j in range(KV):
        kh = _rope(_rms(k[:, j * HD:(j + 1) * HD], kn_ref[...]), cos, sin)
        khs.append(kh)
        vhs.append(v[:, j * HD:(j + 1) * HD])
    scale = 1.0 / np.sqrt(HD).astype(np.float32)
    ctxs = []
    for i in range(H):
        qh = _rope(_rms(q[:, i * HD:(i + 1) * HD], qn_ref[...]), cos, sin)
        kh = khs[i // (H // KV)]
        vh = vhs[i // (H // KV)]
        s_ = jax.lax.dot_general(qh, kh, (((1,), (1,)), ((), ())),
                                 preferred_element_type=jnp.float32) * scale
        s_ = jnp.where(valid, s_, -1e9)
        m = jnp.max(s_, axis=1, keepdims=True)
        ex = jnp.exp(s_ - m)
        p_ = ex / jnp.sum(ex, axis=1, keepdims=True)
        ctxs.append(jnp.dot(p_, vh, preferred_element_type=jnp.float32))
    ctx = jnp.concatenate(ctxs, axis=1)
    o_ref[...] = x + jnp.dot(ctx, wo_ref[...], preferred_element_type=jnp.float32)


def _attn_route_kernel(x_ref, ln1_ref, wq_ref, wk_ref, wv_ref, wo_ref, qn_ref,
                       kn_ref, cos_ref, sin_ref, ln2_ref, wr_ref,
                       xa_ref, xs_ref, sg_ref, offs_ref, cnts_ref, aux_ref):
    _attn_kernel(x_ref, ln1_ref, wq_ref, wk_ref, wv_ref, wo_ref, qn_ref,
                 kn_ref, cos_ref, sin_ref, xa_ref)
    xa = xa_ref[...]
    h2 = _rms(xa, ln2_ref[...])
    logits = jnp.dot(h2, wr_ref[...], preferred_element_type=jnp.float32)
    m = jnp.max(logits, axis=1, keepdims=True)
    ex = jnp.exp(logits - m)
    probs = ex / jnp.sum(ex, axis=1, keepdims=True)
    lane = jax.lax.broadcasted_iota(jnp.int32, (T, E), 1)
    v1 = jnp.max(probs, axis=1, keepdims=True)
    i1 = jnp.min(jnp.where(probs == v1, lane, E), axis=1, keepdims=True)
    oh1 = lane == i1
    probs_m = jnp.where(oh1, -1.0, probs)
    v2 = jnp.max(probs_m, axis=1, keepdims=True)
    i2 = jnp.min(jnp.where(probs_m == v2, lane, E), axis=1, keepdims=True)
    oh2 = lane == i2
    sg = v1 + v2
    g1 = v1 / sg
    g2 = v2 / sg
    ohall = oh1.astype(jnp.float32) + oh2.astype(jnp.float32)
    f_ = jnp.sum(ohall, axis=0, keepdims=True) / (T * TOPK)
    p_ = jnp.sum(probs, axis=0, keepdims=True) / T
    aux_ref[...] = E * jnp.sum(f_ * p_, axis=1, keepdims=True)
    rT = jax.lax.broadcasted_iota(jnp.int32, (T, T), 0)
    cT = jax.lax.broadcasted_iota(jnp.int32, (T, T), 1)
    ltok = (cT < rT).astype(jnp.float32)
    rank = jax.lax.dot_general(ltok, ohall, (((1,), (0,)), ((), ())),
                               preferred_element_type=jnp.float32)
    rE = jax.lax.broadcasted_iota(jnp.int32, (E, E), 0)
    cE = jax.lax.broadcasted_iota(jnp.int32, (E, E), 1)
    lexp_row = (rE < cE).astype(jnp.float32)
    hist_row = jnp.sum(ohall, axis=0, keepdims=True)
    phist_row = jnp.ceil(hist_row * 0.125) * 8.0
    poff_row = jnp.dot(phist_row, lexp_row, preferred_element_type=jnp.float32)
    pos_te = poff_row + rank
    pos1 = jnp.sum(jnp.where(oh1, pos_te, 0.0), axis=1, keepdims=True)
    pos2 = jnp.sum(jnp.where(oh2, pos_te, 0.0), axis=1, keepdims=True)
    pcol = jax.lax.broadcasted_iota(jnp.int32, (T, P), 1)
    m1 = (pcol == pos1.astype(jnp.int32)).astype(jnp.float32)
    m2 = (pcol == pos2.astype(jnp.int32)).astype(jnp.float32)
    smat = m1 + m2
    xs_ref[...] = jax.lax.dot_general(smat, h2, (((0,), (0,)), ((), ())),
                                      preferred_element_type=jnp.float32)
    sg_ref[...] = m1 * g1 + m2 * g2
    offs_ref[...] = (poff_row * 0.125).astype(jnp.int32)
    cnts_ref[...] = hist_row.astype(jnp.int32)


def _moe_kernel(offs_ref, cnts_ref, x_ref, wg_ref, wu_ref, wd_ref, y_ref):
    e = pl.program_id(0)

    @pl.when(e == 0)
    def _():
        y_ref[...] = jnp.zeros((P, D), jnp.float32)

    off8 = offs_ref[0, e]
    cnt = cnts_ref[0, e]
    nt = (cnt + TM - 1) // TM
    rows = jax.lax.broadcasted_iota(jnp.int32, (TM, D), 0)

    def body(j, _):
        base = (off8 + j * (TM // 8)) * 8
        xt = x_ref[pl.ds(base, TM), :]
        g = jnp.dot(xt, wg_ref[0], preferred_element_type=jnp.float32)
        u = jnp.dot(xt, wu_ref[0], preferred_element_type=jnp.float32)
        hid = (g / (1.0 + jnp.exp(-g))) * u
        y = jnp.dot(hid, wd_ref[0], preferred_element_type=jnp.float32)
        mask = rows < cnt - j * TM
        y_ref[pl.ds(base, TM), :] = jnp.where(mask, y, y_ref[pl.ds(base, TM), :])
        return 0

    jax.lax.fori_loop(0, nt, body, 0)


def _comb_kernel(xa_ref, sg_ref, y_ref, o_ref):
    o_ref[...] = xa_ref[...] + jnp.dot(sg_ref[...], y_ref[...],
                                       preferred_element_type=jnp.float32)


def kernel(x, ln1_w, wq, wk, wv, wo, q_norm_w, k_norm_w, ln2_w, w_router,
           wg, wu, wd):
    x2 = x.reshape(T, D)
    cos, sin = _rope_const()
    xa, xs, sgm, offs, cnts, aux = pl.pallas_call(
        _attn_route_kernel,
        out_shape=(
            jax.ShapeDtypeStruct((T, D), jnp.float32),
            jax.ShapeDtypeStruct((P, D), jnp.float32),
            jax.ShapeDtypeStruct((T, P), jnp.float32),
            jax.ShapeDtypeStruct((1, E), jnp.int32),
            jax.ShapeDtypeStruct((1, E), jnp.int32),
            jax.ShapeDtypeStruct((1, 1), jnp.float32),
        ),
    )(x2, ln1_w.reshape(1, D), wq, wk, wv, wo,
      q_norm_w.reshape(1, HD), k_norm_w.reshape(1, HD), cos, sin,
      ln2_w.reshape(1, D), w_router)

    grid_spec = pltpu.PrefetchScalarGridSpec(
        num_scalar_prefetch=2,
        grid=(E,),
        in_specs=[
            pl.BlockSpec((P, D), lambda e, offs, cnts: (0, 0)),
            pl.BlockSpec((1, D, FF), lambda e, offs, cnts: (e, 0, 0)),
            pl.BlockSpec((1, D, FF), lambda e, offs, cnts: (e, 0, 0)),
            pl.BlockSpec((1, FF, D), lambda e, offs, cnts: (e, 0, 0)),
        ],
        out_specs=pl.BlockSpec((P, D), lambda e, offs, cnts: (0, 0)),
    )
    y = pl.pallas_call(
        _moe_kernel,
        grid_spec=grid_spec,
        out_shape=jax.ShapeDtypeStruct((P, D), jnp.float32),
    )(offs, cnts, xs, wg, wu, wd)

    out = pl.pallas_call(
        _comb_kernel,
        out_shape=jax.ShapeDtypeStruct((T, D), jnp.float32),
    )(xa, sgm, y)
    return out.reshape(B, S, D), aux[0, 0]


# speedup vs baseline: 1.0602x; 1.0201x over previous
import numpy as np
import jax
import jax.numpy as jnp
from jax.experimental import pallas as pl
from jax.experimental.pallas import tpu as pltpu

D = 1024; H = 16; KV = 4; HD = D // H; FF = 512; E = 64; TOPK = 2
B = 32; S = 8; T = B * S; A = T * TOPK
EPS = 1e-6
ROPE_THETA = 1000000.0
TM = 64
P = 1024


def _rope_const():
    inv_freq = (1.0 / (ROPE_THETA ** (np.arange(0, HD, 2, dtype=np.float32) / HD)))
    pos = np.arange(S, dtype=np.float32)
    freqs = np.outer(pos, inv_freq)
    emb = np.concatenate([freqs, freqs], axis=-1)
    cos = np.tile(np.cos(emb).astype(np.float32), (B, 1))
    sin = np.tile(np.sin(emb).astype(np.float32), (B, 1))
    return jnp.asarray(cos), jnp.asarray(sin)


def _rms(x, w):
    return x * jax.lax.rsqrt(jnp.mean(x * x, axis=-1, keepdims=True) + EPS) * w


def _rope(x, cos, sin):
    half = HD // 2
    rot = jnp.concatenate([-x[:, half:], x[:, :half]], axis=1)
    return x * cos + rot * sin


def _attn_kernel(x_ref, ln1_ref, wq_ref, wk_ref, wv_ref, wo_ref, qn_ref,
                 kn_ref, cos_ref, sin_ref, o_ref):
    x = x_ref[...]
    h = _rms(x, ln1_ref[...]).astype(jnp.bfloat16)
    q = jnp.dot(h, wq_ref[...].astype(jnp.bfloat16),
                preferred_element_type=jnp.float32)
    k = jnp.dot(h, wk_ref[...].astype(jnp.bfloat16),
                preferred_element_type=jnp.float32)
    v = jnp.dot(h, wv_ref[...].astype(jnp.bfloat16),
                preferred_element_type=jnp.float32)
    cos = cos_ref[...]
    sin = sin_ref[...]
    r = jax.lax.broadcasted_iota(jnp.int32, (T, T), 0)
    c = jax.lax.broadcasted_iota(jnp.int32, (T, T), 1)
    valid = (r // S == c // S) & (c <= r)
    khs, vhs = [], []
    for j in range(KV):
        kh = _rope(_rms(k[:, j * HD:(j + 1) * HD], kn_ref[...]), cos, sin)
        khs.append(kh)
        vhs.append(v[:, j * HD:(j + 1) * HD])
    scale = 1.0 / np.sqrt(HD).astype(np.float32)
    ctxs = []
    for i in range(H):
        qh = _rope(_rms(q[:, i * HD:(i + 1) * HD], qn_ref[...]), cos, sin)
        kh = khs[i // (H // KV)]
        vh = vhs[i // (H // KV)]
        s_ = jax.lax.dot_general(qh, kh, (((1,), (1,)), ((), ())),
                                 preferred_element_type=jnp.float32) * scale
        s_ = jnp.where(valid, s_, -1e9)
        m = jnp.max(s_, axis=1, keepdims=True)
        ex = jnp.exp(s_ - m)
        p_ = ex / jnp.sum(ex, axis=1, keepdims=True)
        ctxs.append(jnp.dot(p_, vh, preferred_element_type=jnp.float32))
    ctx = jnp.concatenate(ctxs, axis=1).astype(jnp.bfloat16)
    o_ref[...] = x + jnp.dot(ctx, wo_ref[...].astype(jnp.bfloat16),
                             preferred_element_type=jnp.float32)


def _attn_route_kernel(x_ref, ln1_ref, wq_ref, wk_ref, wv_ref, wo_ref, qn_ref,
                       kn_ref, cos_ref, sin_ref, ln2_ref, wr_ref,
                       xa_ref, xs_ref, sg_ref, offs_ref, cnts_ref, aux_ref):
    _attn_kernel(x_ref, ln1_ref, wq_ref, wk_ref, wv_ref, wo_ref, qn_ref,
                 kn_ref, cos_ref, sin_ref, xa_ref)
    xa = xa_ref[...]
    h2 = _rms(xa, ln2_ref[...])
    logits = jnp.dot(h2, wr_ref[...], preferred_element_type=jnp.float32)
    m = jnp.max(logits, axis=1, keepdims=True)
    ex = jnp.exp(logits - m)
    probs = ex / jnp.sum(ex, axis=1, keepdims=True)
    lane = jax.lax.broadcasted_iota(jnp.int32, (T, E), 1)
    v1 = jnp.max(probs, axis=1, keepdims=True)
    i1 = jnp.min(jnp.where(probs == v1, lane, E), axis=1, keepdims=True)
    oh1 = lane == i1
    probs_m = jnp.where(oh1, -1.0, probs)
    v2 = jnp.max(probs_m, axis=1, keepdims=True)
    i2 = jnp.min(jnp.where(probs_m == v2, lane, E), axis=1, keepdims=True)
    oh2 = lane == i2
    sg = v1 + v2
    g1 = v1 / sg
    g2 = v2 / sg
    ohall = oh1.astype(jnp.float32) + oh2.astype(jnp.float32)
    f_ = jnp.sum(ohall, axis=0, keepdims=True) / (T * TOPK)
    p_ = jnp.sum(probs, axis=0, keepdims=True) / T
    aux_ref[...] = E * jnp.sum(f_ * p_, axis=1, keepdims=True)
    rT = jax.lax.broadcasted_iota(jnp.int32, (T, T), 0)
    cT = jax.lax.broadcasted_iota(jnp.int32, (T, T), 1)
    ltok = (cT < rT).astype(jnp.float32)
    rank = jax.lax.dot_general(ltok, ohall, (((1,), (0,)), ((), ())),
                               preferred_element_type=jnp.float32)
    rE = jax.lax.broadcasted_iota(jnp.int32, (E, E), 0)
    cE = jax.lax.broadcasted_iota(jnp.int32, (E, E), 1)
    lexp_row = (rE < cE).astype(jnp.float32)
    hist_row = jnp.sum(ohall, axis=0, keepdims=True)
    phist_row = jnp.ceil(hist_row * 0.125) * 8.0
    poff_row = jnp.dot(phist_row, lexp_row, preferred_element_type=jnp.float32)
    pos_te = poff_row + rank
    pos1 = jnp.sum(jnp.where(oh1, pos_te, 0.0), axis=1, keepdims=True)
    pos2 = jnp.sum(jnp.where(oh2, pos_te, 0.0), axis=1, keepdims=True)
    pcol = jax.lax.broadcasted_iota(jnp.int32, (T, P), 1)
    m1 = (pcol == pos1.astype(jnp.int32)).astype(jnp.float32)
    m2 = (pcol == pos2.astype(jnp.int32)).astype(jnp.float32)
    smat = (m1 + m2).astype(jnp.bfloat16)
    xs_ref[...] = jax.lax.dot_general(smat, h2.astype(jnp.bfloat16),
                                      (((0,), (0,)), ((), ())),
                                      preferred_element_type=jnp.float32)
    sg_ref[...] = m1 * g1 + m2 * g2
    offs_ref[...] = (poff_row * 0.125).astype(jnp.int32)
    cnts_ref[...] = hist_row.astype(jnp.int32)


def _moe_kernel(offs_ref, cnts_ref, x_ref, xa_ref, sg_ref,
                wg_ref, wu_ref, wd_ref, o_ref, y_ref):
    e = pl.program_id(0)

    @pl.when(e == 0)
    def _():
        y_ref[...] = jnp.zeros((P, D), jnp.float32)

    off8 = offs_ref[0, e]
    cnt = cnts_ref[0, e]
    nt = (cnt + TM - 1) // TM
    rows = jax.lax.broadcasted_iota(jnp.int32, (TM, D), 0)

    def body(j, _):
        base = (off8 + j * (TM // 8)) * 8
        xt = x_ref[pl.ds(base, TM), :]
        g = jnp.dot(xt, wg_ref[0], preferred_element_type=jnp.float32)
        u = jnp.dot(xt, wu_ref[0], preferred_element_type=jnp.float32)
        hid = (g / (1.0 + jnp.exp(-g))) * u
        y = jnp.dot(hid, wd_ref[0], preferred_element_type=jnp.float32)
        mask = rows < cnt - j * TM
        y_ref[pl.ds(base, TM), :] = jnp.where(mask, y, y_ref[pl.ds(base, TM), :])
        return 0

    jax.lax.fori_loop(0, nt, body, 0)

    @pl.when(e == E - 1)
    def _():
        sgb = sg_ref[...].astype(jnp.bfloat16)
        yb = y_ref[...].astype(jnp.bfloat16)
        o_ref[...] = xa_ref[...] + jnp.dot(sgb, yb,
                                           preferred_element_type=jnp.float32)


def _comb_kernel(xa_ref, sg_ref, y_ref, o_ref):
    o_ref[...] = xa_ref[...] + jnp.dot(sg_ref[...], y_ref[...],
                                       preferred_element_type=jnp.float32)


def kernel(x, ln1_w, wq, wk, wv, wo, q_norm_w, k_norm_w, ln2_w, w_router,
           wg, wu, wd):
    x2 = x.reshape(T, D)
    cos, sin = _rope_const()
    xa, xs, sgm, offs, cnts, aux = pl.pallas_call(
        _attn_route_kernel,
        out_shape=(
            jax.ShapeDtypeStruct((T, D), jnp.float32),
            jax.ShapeDtypeStruct((P, D), jnp.float32),
            jax.ShapeDtypeStruct((T, P), jnp.float32),
            jax.ShapeDtypeStruct((1, E), jnp.int32),
            jax.ShapeDtypeStruct((1, E), jnp.int32),
            jax.ShapeDtypeStruct((1, 1), jnp.float32),
        ),
    )(x2, ln1_w.reshape(1, D), wq, wk, wv, wo,
      q_norm_w.reshape(1, HD), k_norm_w.reshape(1, HD), cos, sin,
      ln2_w.reshape(1, D), w_router)

    grid_spec = pltpu.PrefetchScalarGridSpec(
        num_scalar_prefetch=2,
        grid=(E,),
        in_specs=[
            pl.BlockSpec((P, D), lambda e, offs, cnts: (0, 0)),
            pl.BlockSpec((T, D), lambda e, offs, cnts: (0, 0)),
            pl.BlockSpec((T, P), lambda e, offs, cnts: (0, 0)),
            pl.BlockSpec((1, D, FF), lambda e, offs, cnts: (e, 0, 0)),
            pl.BlockSpec((1, D, FF), lambda e, offs, cnts: (e, 0, 0)),
            pl.BlockSpec((1, FF, D), lambda e, offs, cnts: (e, 0, 0)),
        ],
        out_specs=pl.BlockSpec((T, D), lambda e, offs, cnts: (0, 0)),
        scratch_shapes=[pltpu.VMEM((P, D), jnp.float32)],
    )
    out = pl.pallas_call(
        _moe_kernel,
        grid_spec=grid_spec,
        out_shape=jax.ShapeDtypeStruct((T, D), jnp.float32),
    )(offs, cnts, xs, xa, sgm, wg, wu, wd)
    return out.reshape(B, S, D), aux[0, 0]


# speedup vs baseline: 1.0968x; 1.0346x over previous
import numpy as np
import jax
import jax.numpy as jnp
from jax.experimental import pallas as pl
from jax.experimental.pallas import tpu as pltpu

D = 1024; H = 16; KV = 4; HD = D // H; FF = 512; E = 64; TOPK = 2
B = 32; S = 8; T = B * S; A = T * TOPK
EPS = 1e-6
ROPE_THETA = 1000000.0
TM = 64
P = 1024


def _rope_const():
    inv_freq = (1.0 / (ROPE_THETA ** (np.arange(0, HD, 2, dtype=np.float32) / HD)))
    pos = np.arange(S, dtype=np.float32)
    freqs = np.outer(pos, inv_freq)
    emb = np.concatenate([freqs, freqs], axis=-1)
    cos = np.tile(np.cos(emb).astype(np.float32), (B, H))
    sin = np.tile(np.sin(emb).astype(np.float32), (B, H))
    bd = np.zeros((D, D), np.float32)
    pm = np.zeros((D, D), np.float32)
    half = HD // 2
    for hh in range(H):
        o = hh * HD
        bd[o:o + HD, o:o + HD] = 1.0
        for j in range(half):
            pm[o + half + j, o + j] = -1.0
            pm[o + j, o + half + j] = 1.0
    return (jnp.asarray(cos), jnp.asarray(sin),
            jnp.asarray(bd, jnp.bfloat16), jnp.asarray(pm, jnp.bfloat16))


def _rms(x, w):
    return x * jax.lax.rsqrt(jnp.mean(x * x, axis=-1, keepdims=True) + EPS) * w


def _rope(x, cos, sin):
    half = HD // 2
    rot = jnp.concatenate([-x[:, half:], x[:, :half]], axis=1)
    return x * cos + rot * sin


def _attn_kernel(x_ref, ln1_ref, wq_ref, wk_ref, wv_ref, wo_ref, qn_ref,
                 kn_ref, cos_ref, sin_ref, bd_ref, pm_ref, o_ref):
    x = x_ref[...]
    h = _rms(x, ln1_ref[...]).astype(jnp.bfloat16)
    q = jnp.dot(h, wq_ref[...].astype(jnp.bfloat16),
                preferred_element_type=jnp.float32)
    k = jnp.dot(h, wk_ref[...].astype(jnp.bfloat16),
                preferred_element_type=jnp.float32)
    v = jnp.dot(h, wv_ref[...].astype(jnp.bfloat16),
                preferred_element_type=jnp.float32)
    cos = cos_ref[...]
    sin = sin_ref[...]
    bd = bd_ref[...]
    pm = pm_ref[...]
    KD = KV * HD
    ssq = jnp.dot((q * q).astype(jnp.bfloat16), bd,
                  preferred_element_type=jnp.float32)
    qn = q * jax.lax.rsqrt(ssq * (1.0 / HD) + EPS) * qn_ref[...]
    qr = (qn * cos + jnp.dot(qn.astype(jnp.bfloat16), pm,
                             preferred_element_type=jnp.float32) * sin)
    ssk = jnp.dot((k * k).astype(jnp.bfloat16), bd[:KD, :KD],
                  preferred_element_type=jnp.float32)
    kn = k * jax.lax.rsqrt(ssk * (1.0 / HD) + EPS) * kn_ref[...]
    kr = (kn * cos[:, :KD] + jnp.dot(kn.astype(jnp.bfloat16), pm[:KD, :KD],
                                     preferred_element_type=jnp.float32)
          * sin[:, :KD])
    r = jax.lax.broadcasted_iota(jnp.int32, (T, T), 0)
    c = jax.lax.broadcasted_iota(jnp.int32, (T, T), 1)
    valid = (r // S == c // S) & (c <= r)
    scale = 1.0 / np.sqrt(HD).astype(np.float32)
    ctxs = []
    for i in range(H):
        qh = qr[:, i * HD:(i + 1) * HD]
        j = i // (H // KV)
        kh = kr[:, j * HD:(j + 1) * HD]
        vh = v[:, j * HD:(j + 1) * HD]
        s_ = jax.lax.dot_general(qh, kh, (((1,), (1,)), ((), ())),
                                 preferred_element_type=jnp.float32) * scale
        s_ = jnp.where(valid, s_, -1e9)
        m = jnp.max(s_, axis=1, keepdims=True)
        ex = jnp.exp(s_ - m)
        p_ = ex * (1.0 / jnp.sum(ex, axis=1, keepdims=True))
        ctxs.append(jnp.dot(p_, vh, preferred_element_type=jnp.float32))
    ctx = jnp.concatenate(ctxs, axis=1).astype(jnp.bfloat16)
    o_ref[...] = x + jnp.dot(ctx, wo_ref[...].astype(jnp.bfloat16),
                             preferred_element_type=jnp.float32)


def _attn_route_kernel(x_ref, ln1_ref, wq_ref, wk_ref, wv_ref, wo_ref, qn_ref,
                       kn_ref, cos_ref, sin_ref, bd_ref, pm_ref, ln2_ref,
                       wr_ref,
                       xa_ref, xs_ref, sg_ref, offs_ref, cnts_ref, aux_ref):
    _attn_kernel(x_ref, ln1_ref, wq_ref, wk_ref, wv_ref, wo_ref, qn_ref,
                 kn_ref, cos_ref, sin_ref, bd_ref, pm_ref, xa_ref)
    xa = xa_ref[...]
    h2 = _rms(xa, ln2_ref[...])
    logits = jnp.dot(h2, wr_ref[...], preferred_element_type=jnp.float32)
    m = jnp.max(logits, axis=1, keepdims=True)
    ex = jnp.exp(logits - m)
    probs = ex * (1.0 / jnp.sum(ex, axis=1, keepdims=True))
    lane = jax.lax.broadcasted_iota(jnp.int32, (T, E), 1)
    v1 = jnp.max(probs, axis=1, keepdims=True)
    i1 = jnp.min(jnp.where(probs == v1, lane, E), axis=1, keepdims=True)
    oh1 = lane == i1
    probs_m = jnp.where(oh1, -1.0, probs)
    v2 = jnp.max(probs_m, axis=1, keepdims=True)
    i2 = jnp.min(jnp.where(probs_m == v2, lane, E), axis=1, keepdims=True)
    oh2 = lane == i2
    sg = v1 + v2
    g1 = v1 / sg
    g2 = v2 / sg
    ohall = oh1.astype(jnp.float32) + oh2.astype(jnp.float32)
    f_ = jnp.sum(ohall, axis=0, keepdims=True) / (T * TOPK)
    p_ = jnp.sum(probs, axis=0, keepdims=True) / T
    aux_ref[...] = E * jnp.sum(f_ * p_, axis=1, keepdims=True)
    rT = jax.lax.broadcasted_iota(jnp.int32, (T, T), 0)
    cT = jax.lax.broadcasted_iota(jnp.int32, (T, T), 1)
    ltok = (cT < rT).astype(jnp.float32)
    rank = jax.lax.dot_general(ltok, ohall, (((1,), (0,)), ((), ())),
                               preferred_element_type=jnp.float32)
    rE = jax.lax.broadcasted_iota(jnp.int32, (E, E), 0)
    cE = jax.lax.broadcasted_iota(jnp.int32, (E, E), 1)
    lexp_row = (rE < cE).astype(jnp.float32)
    hist_row = jnp.sum(ohall, axis=0, keepdims=True)
    phist_row = jnp.ceil(hist_row * 0.125) * 8.0
    poff_row = jnp.dot(phist_row, lexp_row, preferred_element_type=jnp.float32)
    pos_te = poff_row + rank
    pos1 = jnp.sum(jnp.where(oh1, pos_te, 0.0), axis=1, keepdims=True)
    pos2 = jnp.sum(jnp.where(oh2, pos_te, 0.0), axis=1, keepdims=True)
    pcol = jax.lax.broadcasted_iota(jnp.int32, (T, P), 1)
    m1 = (pcol == pos1.astype(jnp.int32)).astype(jnp.float32)
    m2 = (pcol == pos2.astype(jnp.int32)).astype(jnp.float32)
    smat = (m1 + m2).astype(jnp.bfloat16)
    xs_ref[...] = jax.lax.dot_general(smat, h2.astype(jnp.bfloat16),
                                      (((0,), (0,)), ((), ())),
                                      preferred_element_type=jnp.float32)
    sg_ref[...] = m1 * g1 + m2 * g2
    offs_ref[...] = (poff_row * 0.125).astype(jnp.int32)
    cnts_ref[...] = hist_row.astype(jnp.int32)


def _moe_kernel(offs_ref, cnts_ref, x_ref, xa_ref, sg_ref,
                wg_ref, wu_ref, wd_ref, o_ref, y_ref):
    e = pl.program_id(0)

    @pl.when(e == 0)
    def _():
        y_ref[...] = jnp.zeros((P, D), jnp.float32)

    off8 = offs_ref[0, e]
    cnt = cnts_ref[0, e]
    nt = (cnt + TM - 1) // TM
    rows = jax.lax.broadcasted_iota(jnp.int32, (TM, D), 0)

    def body(j, _):
        base = (off8 + j * (TM // 8)) * 8
        xt = x_ref[pl.ds(base, TM), :]
        g = jnp.dot(xt, wg_ref[0], preferred_element_type=jnp.float32)
        u = jnp.dot(xt, wu_ref[0], preferred_element_type=jnp.float32)
        hid = (g / (1.0 + jnp.exp(-g))) * u
        y = jnp.dot(hid, wd_ref[0], preferred_element_type=jnp.float32)
        mask = rows < cnt - j * TM
        y_ref[pl.ds(base, TM), :] = jnp.where(mask, y, y_ref[pl.ds(base, TM), :])
        return 0

    jax.lax.fori_loop(0, nt, body, 0)

    @pl.when(e == E - 1)
    def _():
        sgb = sg_ref[...].astype(jnp.bfloat16)
        yb = y_ref[...].astype(jnp.bfloat16)
        o_ref[...] = xa_ref[...] + jnp.dot(sgb, yb,
                                           preferred_element_type=jnp.float32)


def _comb_kernel(xa_ref, sg_ref, y_ref, o_ref):
    o_ref[...] = xa_ref[...] + jnp.dot(sg_ref[...], y_ref[...],
                                       preferred_element_type=jnp.float32)


def kernel(x, ln1_w, wq, wk, wv, wo, q_norm_w, k_norm_w, ln2_w, w_router,
           wg, wu, wd):
    x2 = x.reshape(T, D)
    cos, sin, bd, pm = _rope_const()
    xa, xs, sgm, offs, cnts, aux = pl.pallas_call(
        _attn_route_kernel,
        out_shape=(
            jax.ShapeDtypeStruct((T, D), jnp.float32),
            jax.ShapeDtypeStruct((P, D), jnp.float32),
            jax.ShapeDtypeStruct((T, P), jnp.float32),
            jax.ShapeDtypeStruct((1, E), jnp.int32),
            jax.ShapeDtypeStruct((1, E), jnp.int32),
            jax.ShapeDtypeStruct((1, 1), jnp.float32),
        ),
    )(x2, ln1_w.reshape(1, D), wq, wk, wv, wo,
      jnp.tile(q_norm_w.reshape(1, HD), (1, H)),
      jnp.tile(k_norm_w.reshape(1, HD), (1, KV)), cos, sin, bd, pm,
      ln2_w.reshape(1, D), w_router)

    grid_spec = pltpu.PrefetchScalarGridSpec(
        num_scalar_prefetch=2,
        grid=(E,),
        in_specs=[
            pl.BlockSpec((P, D), lambda e, offs, cnts: (0, 0)),
            pl.BlockSpec((T, D), lambda e, offs, cnts: (0, 0)),
            pl.BlockSpec((T, P), lambda e, offs, cnts: (0, 0)),
            pl.BlockSpec((1, D, FF), lambda e, offs, cnts: (e, 0, 0)),
            pl.BlockSpec((1, D, FF), lambda e, offs, cnts: (e, 0, 0)),
            pl.BlockSpec((1, FF, D), lambda e, offs, cnts: (e, 0, 0)),
        ],
        out_specs=pl.BlockSpec((T, D), lambda e, offs, cnts: (0, 0)),
        scratch_shapes=[pltpu.VMEM((P, D), jnp.float32)],
    )
    out = pl.pallas_call(
        _moe_kernel,
        grid_spec=grid_spec,
        out_shape=jax.ShapeDtypeStruct((T, D), jnp.float32),
    )(offs, cnts, xs, xa, sgm, wg, wu, wd)
    return out.reshape(B, S, D), aux[0, 0]


# speedup vs baseline: 1.1625x; 1.0599x over previous
import numpy as np
import jax
import jax.numpy as jnp
from jax.experimental import pallas as pl
from jax.experimental.pallas import tpu as pltpu

D = 1024; H = 16; KV = 4; HD = D // H; FF = 512; E = 64; TOPK = 2
B = 32; S = 8; T = B * S; A = T * TOPK
EPS = 1e-6
ROPE_THETA = 1000000.0
TM = 64
EB = 2
P = 1024


def _rope_const():
    inv_freq = (1.0 / (ROPE_THETA ** (np.arange(0, HD, 2, dtype=np.float32) / HD)))
    pos = np.arange(S, dtype=np.float32)
    freqs = np.outer(pos, inv_freq)
    emb = np.concatenate([freqs, freqs], axis=-1)
    cos = np.tile(np.cos(emb).astype(np.float32), (B, H))
    sin = np.tile(np.sin(emb).astype(np.float32), (B, H))
    bd = np.zeros((D, D), np.float32)
    pm = np.zeros((D, D), np.float32)
    half = HD // 2
    for hh in range(H):
        o = hh * HD
        bd[o:o + HD, o:o + HD] = 1.0
        for j in range(half):
            pm[o + half + j, o + j] = -1.0
            pm[o + j, o + half + j] = 1.0
    return (jnp.asarray(cos), jnp.asarray(sin),
            jnp.asarray(bd, jnp.bfloat16), jnp.asarray(pm, jnp.bfloat16))


def _rms(x, w):
    return x * jax.lax.rsqrt(jnp.mean(x * x, axis=-1, keepdims=True) + EPS) * w


def _rope(x, cos, sin):
    half = HD // 2
    rot = jnp.concatenate([-x[:, half:], x[:, :half]], axis=1)
    return x * cos + rot * sin


def _attn_kernel(x_ref, ln1_ref, wq_ref, wk_ref, wv_ref, wo_ref, qn_ref,
                 kn_ref, cos_ref, sin_ref, bd_ref, pm_ref, o_ref):
    x = x_ref[...]
    h = _rms(x, ln1_ref[...]).astype(jnp.bfloat16)
    q = jnp.dot(h, wq_ref[...].astype(jnp.bfloat16),
                preferred_element_type=jnp.float32)
    k = jnp.dot(h, wk_ref[...].astype(jnp.bfloat16),
                preferred_element_type=jnp.float32)
    v = jnp.dot(h, wv_ref[...].astype(jnp.bfloat16),
                preferred_element_type=jnp.float32)
    cos = cos_ref[...]
    sin = sin_ref[...]
    bd = bd_ref[...]
    pm = pm_ref[...]
    KD = KV * HD
    ssq = jnp.dot((q * q).astype(jnp.bfloat16), bd,
                  preferred_element_type=jnp.float32)
    qn = q * jax.lax.rsqrt(ssq * (1.0 / HD) + EPS) * qn_ref[...]
    qr = (qn * cos + jnp.dot(qn.astype(jnp.bfloat16), pm,
                             preferred_element_type=jnp.float32) * sin)
    ssk = jnp.dot((k * k).astype(jnp.bfloat16), bd[:KD, :KD],
                  preferred_element_type=jnp.float32)
    kn = k * jax.lax.rsqrt(ssk * (1.0 / HD) + EPS) * kn_ref[...]
    kr = (kn * cos[:, :KD] + jnp.dot(kn.astype(jnp.bfloat16), pm[:KD, :KD],
                                     preferred_element_type=jnp.float32)
          * sin[:, :KD])
    r = jax.lax.broadcasted_iota(jnp.int32, (T, T), 0)
    c = jax.lax.broadcasted_iota(jnp.int32, (T, T), 1)
    valid = (r // S == c // S) & (c <= r)
    scale = 1.0 / np.sqrt(HD).astype(np.float32)
    ctxs = []
    for i in range(H):
        qh = qr[:, i * HD:(i + 1) * HD]
        j = i // (H // KV)
        kh = kr[:, j * HD:(j + 1) * HD]
        vh = v[:, j * HD:(j + 1) * HD]
        s_ = jax.lax.dot_general(qh, kh, (((1,), (1,)), ((), ())),
                                 preferred_element_type=jnp.float32) * scale
        s_ = jnp.where(valid, s_, -1e9)
        m = jnp.max(s_, axis=1, keepdims=True)
        ex = jnp.exp(s_ - m)
        p_ = ex * (1.0 / jnp.sum(ex, axis=1, keepdims=True))
        ctxs.append(jnp.dot(p_, vh, preferred_element_type=jnp.float32))
    ctx = jnp.concatenate(ctxs, axis=1).astype(jnp.bfloat16)
    o_ref[...] = x + jnp.dot(ctx, wo_ref[...].astype(jnp.bfloat16),
                             preferred_element_type=jnp.float32)


def _attn_route_kernel(x_ref, ln1_ref, wq_ref, wk_ref, wv_ref, wo_ref, qn_ref,
                       kn_ref, cos_ref, sin_ref, bd_ref, pm_ref, ln2_ref,
                       wr_ref,
                       xa_ref, xs_ref, sg_ref, offs_ref, cnts_ref, aux_ref):
    _attn_kernel(x_ref, ln1_ref, wq_ref, wk_ref, wv_ref, wo_ref, qn_ref,
                 kn_ref, cos_ref, sin_ref, bd_ref, pm_ref, xa_ref)
    xa = xa_ref[...]
    h2 = _rms(xa, ln2_ref[...])
    logits = jnp.dot(h2, wr_ref[...], preferred_element_type=jnp.float32)
    m = jnp.max(logits, axis=1, keepdims=True)
    ex = jnp.exp(logits - m)
    probs = ex * (1.0 / jnp.sum(ex, axis=1, keepdims=True))
    lane = jax.lax.broadcasted_iota(jnp.int32, (T, E), 1)
    v1 = jnp.max(probs, axis=1, keepdims=True)
    i1 = jnp.min(jnp.where(probs == v1, lane, E), axis=1, keepdims=True)
    oh1 = lane == i1
    probs_m = jnp.where(oh1, -1.0, probs)
    v2 = jnp.max(probs_m, axis=1, keepdims=True)
    i2 = jnp.min(jnp.where(probs_m == v2, lane, E), axis=1, keepdims=True)
    oh2 = lane == i2
    sg = v1 + v2
    g1 = v1 / sg
    g2 = v2 / sg
    ohall = oh1.astype(jnp.float32) + oh2.astype(jnp.float32)
    f_ = jnp.sum(ohall, axis=0, keepdims=True) / (T * TOPK)
    p_ = jnp.sum(probs, axis=0, keepdims=True) / T
    aux_ref[...] = E * jnp.sum(f_ * p_, axis=1, keepdims=True)
    rT = jax.lax.broadcasted_iota(jnp.int32, (T, T), 0)
    cT = jax.lax.broadcasted_iota(jnp.int32, (T, T), 1)
    ltok = (cT < rT).astype(jnp.float32)
    rank = jax.lax.dot_general(ltok, ohall, (((1,), (0,)), ((), ())),
                               preferred_element_type=jnp.float32)
    rE = jax.lax.broadcasted_iota(jnp.int32, (E, E), 0)
    cE = jax.lax.broadcasted_iota(jnp.int32, (E, E), 1)
    lexp_row = (rE < cE).astype(jnp.float32)
    hist_row = jnp.sum(ohall, axis=0, keepdims=True)
    phist_row = jnp.ceil(hist_row * 0.125) * 8.0
    poff_row = jnp.dot(phist_row, lexp_row, preferred_element_type=jnp.float32)
    pos_te = poff_row + rank
    pos1 = jnp.sum(jnp.where(oh1, pos_te, 0.0), axis=1, keepdims=True)
    pos2 = jnp.sum(jnp.where(oh2, pos_te, 0.0), axis=1, keepdims=True)
    pcol = jax.lax.broadcasted_iota(jnp.int32, (T, P), 1)
    m1 = (pcol == pos1.astype(jnp.int32)).astype(jnp.float32)
    m2 = (pcol == pos2.astype(jnp.int32)).astype(jnp.float32)
    smat = (m1 + m2).astype(jnp.bfloat16)
    xs_ref[...] = jax.lax.dot_general(smat, h2.astype(jnp.bfloat16),
                                      (((0,), (0,)), ((), ())),
                                      preferred_element_type=jnp.float32)
    sg_ref[...] = m1 * g1 + m2 * g2
    offs_ref[...] = (poff_row * 0.125).astype(jnp.int32)
    cnts_ref[...] = hist_row.astype(jnp.int32)


def _moe_kernel(offs_ref, cnts_ref, x_ref, xa_ref, sg_ref,
                wg_ref, wu_ref, wd_ref, o_ref, y_ref):
    eb = pl.program_id(0)

    @pl.when(eb == 0)
    def _():
        y_ref[...] = jnp.zeros((P, D), jnp.float32)

    rows = jax.lax.broadcasted_iota(jnp.int32, (TM, D), 0)

    for sub in range(EB):
        e = eb * EB + sub
        off8 = offs_ref[0, e]
        cnt = cnts_ref[0, e]
        nt = (cnt + TM - 1) // TM

        def body(j, _, off8=off8, cnt=cnt, sub=sub):
            base = (off8 + j * (TM // 8)) * 8
            xt = x_ref[pl.ds(base, TM), :]
            g = jnp.dot(xt, wg_ref[sub], preferred_element_type=jnp.float32)
            u = jnp.dot(xt, wu_ref[sub], preferred_element_type=jnp.float32)
            hid = (g / (1.0 + jnp.exp(-g))) * u
            y = jnp.dot(hid, wd_ref[sub], preferred_element_type=jnp.float32)
            mask = rows < cnt - j * TM
            y_ref[pl.ds(base, TM), :] = jnp.where(mask, y,
                                                  y_ref[pl.ds(base, TM), :])
            return 0

        jax.lax.fori_loop(0, nt, body, 0)

    @pl.when(eb == E // EB - 1)
    def _():
        sgb = sg_ref[...].astype(jnp.bfloat16)
        yb = y_ref[...].astype(jnp.bfloat16)
        o_ref[...] = xa_ref[...] + jnp.dot(sgb, yb,
                                           preferred_element_type=jnp.float32)


def _comb_kernel(xa_ref, sg_ref, y_ref, o_ref):
    o_ref[...] = xa_ref[...] + jnp.dot(sg_ref[...], y_ref[...],
                                       preferred_element_type=jnp.float32)


def kernel(x, ln1_w, wq, wk, wv, wo, q_norm_w, k_norm_w, ln2_w, w_router,
           wg, wu, wd):
    x2 = x.reshape(T, D)
    cos, sin, bd, pm = _rope_const()
    xa, xs, sgm, offs, cnts, aux = pl.pallas_call(
        _attn_route_kernel,
        out_shape=(
            jax.ShapeDtypeStruct((T, D), jnp.float32),
            jax.ShapeDtypeStruct((P, D), jnp.float32),
            jax.ShapeDtypeStruct((T, P), jnp.float32),
            jax.ShapeDtypeStruct((1, E), jnp.int32),
            jax.ShapeDtypeStruct((1, E), jnp.int32),
            jax.ShapeDtypeStruct((1, 1), jnp.float32),
        ),
    )(x2, ln1_w.reshape(1, D), wq, wk, wv, wo,
      jnp.tile(q_norm_w.reshape(1, HD), (1, H)),
      jnp.tile(k_norm_w.reshape(1, HD), (1, KV)), cos, sin, bd, pm,
      ln2_w.reshape(1, D), w_router)

    grid_spec = pltpu.PrefetchScalarGridSpec(
        num_scalar_prefetch=2,
        grid=(E // EB,),
        in_specs=[
            pl.BlockSpec((P, D), lambda e, offs, cnts: (0, 0)),
            pl.BlockSpec((T, D), lambda e, offs, cnts: (0, 0)),
            pl.BlockSpec((T, P), lambda e, offs, cnts: (0, 0)),
            pl.BlockSpec((EB, D, FF), lambda e, offs, cnts: (e, 0, 0)),
            pl.BlockSpec((EB, D, FF), lambda e, offs, cnts: (e, 0, 0)),
            pl.BlockSpec((EB, FF, D), lambda e, offs, cnts: (e, 0, 0)),
        ],
        out_specs=pl.BlockSpec((T, D), lambda e, offs, cnts: (0, 0)),
        scratch_shapes=[pltpu.VMEM((P, D), jnp.float32)],
    )
    out = pl.pallas_call(
        _moe_kernel,
        grid_spec=grid_spec,
        out_shape=jax.ShapeDtypeStruct((T, D), jnp.float32),
    )(offs, cnts, xs, xa, sgm, wg, wu, wd)
    return out.reshape(B, S, D), aux[0, 0]


# speedup vs baseline: 1.1640x; 1.0013x over previous
import numpy as np
import jax
import jax.numpy as jnp
from jax.experimental import pallas as pl
from jax.experimental.pallas import tpu as pltpu

D = 1024; H = 16; KV = 4; HD = D // H; FF = 512; E = 64; TOPK = 2
B = 32; S = 8; T = B * S; A = T * TOPK
EPS = 1e-6
ROPE_THETA = 1000000.0
TM = 64
EB = 2
P = 1024


def _rope_const():
    inv_freq = (1.0 / (ROPE_THETA ** (np.arange(0, HD, 2, dtype=np.float32) / HD)))
    pos = np.arange(S, dtype=np.float32)
    freqs = np.outer(pos, inv_freq)
    emb = np.concatenate([freqs, freqs], axis=-1)
    cos = np.tile(np.cos(emb).astype(np.float32), (B, H))
    sin = np.tile(np.sin(emb).astype(np.float32), (B, H))
    bd = np.zeros((D, D), np.float32)
    pm = np.zeros((D, D), np.float32)
    half = HD // 2
    for hh in range(H):
        o = hh * HD
        bd[o:o + HD, o:o + HD] = 1.0
        for j in range(half):
            pm[o + half + j, o + j] = -1.0
            pm[o + j, o + half + j] = 1.0
    return (jnp.asarray(cos), jnp.asarray(sin),
            jnp.asarray(bd, jnp.bfloat16), jnp.asarray(pm, jnp.bfloat16))


def _rms(x, w):
    return x * jax.lax.rsqrt(jnp.mean(x * x, axis=-1, keepdims=True) + EPS) * w


def _attn_kernel(x_ref, ln1_ref, wq_ref, wk_ref, wv_ref, wo_ref, qn_ref,
                 kn_ref, cos_ref, sin_ref, bd_ref, pm_ref, o_ref):
    x = x_ref[...]
    h = _rms(x, ln1_ref[...]).astype(jnp.bfloat16)
    q = jnp.dot(h, wq_ref[...].astype(jnp.bfloat16),
                preferred_element_type=jnp.float32)
    k = jnp.dot(h, wk_ref[...].astype(jnp.bfloat16),
                preferred_element_type=jnp.float32)
    v = jnp.dot(h, wv_ref[...].astype(jnp.bfloat16),
                preferred_element_type=jnp.float32)
    cos = cos_ref[...]
    sin = sin_ref[...]
    bd = bd_ref[...]
    pm = pm_ref[...]
    KD = KV * HD
    ssq = jnp.dot((q * q).astype(jnp.bfloat16), bd,
                  preferred_element_type=jnp.float32)
    qn = q * jax.lax.rsqrt(ssq * (1.0 / HD) + EPS) * qn_ref[...]
    qr = (qn * cos + jnp.dot(qn.astype(jnp.bfloat16), pm,
                             preferred_element_type=jnp.float32) * sin)
    ssk = jnp.dot((k * k).astype(jnp.bfloat16), bd[:KD, :KD],
                  preferred_element_type=jnp.float32)
    kn = k * jax.lax.rsqrt(ssk * (1.0 / HD) + EPS) * kn_ref[...]
    kr = (kn * cos[:, :KD] + jnp.dot(kn.astype(jnp.bfloat16), pm[:KD, :KD],
                                     preferred_element_type=jnp.float32)
          * sin[:, :KD])
    r = jax.lax.broadcasted_iota(jnp.int32, (T, T), 0)
    c = jax.lax.broadcasted_iota(jnp.int32, (T, T), 1)
    valid = (r // S == c // S) & (c <= r)
    scale = 1.0 / np.sqrt(HD).astype(np.float32)
    ctxs = []
    for i in range(H):
        qh = qr[:, i * HD:(i + 1) * HD]
        j = i // (H // KV)
        kh = kr[:, j * HD:(j + 1) * HD]
        vh = v[:, j * HD:(j + 1) * HD]
        s_ = jax.lax.dot_general(qh, kh, (((1,), (1,)), ((), ())),
                                 preferred_element_type=jnp.float32) * scale
        s_ = jnp.where(valid, s_, -1e9)
        m = jnp.max(s_, axis=1, keepdims=True)
        ex = jnp.exp(s_ - m)
        p_ = ex * (1.0 / jnp.sum(ex, axis=1, keepdims=True))
        ctxs.append(jnp.dot(p_, vh, preferred_element_type=jnp.float32))
    ctx = jnp.concatenate(ctxs, axis=1).astype(jnp.bfloat16)
    o_ref[...] = x + jnp.dot(ctx, wo_ref[...].astype(jnp.bfloat16),
                             preferred_element_type=jnp.float32)


def _attn_route_kernel(x_ref, ln1_ref, wq_ref, wk_ref, wv_ref, wo_ref, qn_ref,
                       kn_ref, cos_ref, sin_ref, bd_ref, pm_ref, ln2_ref,
                       wr_ref,
                       xa_ref, xs_ref, sg_ref, offs_ref, cnts_ref, aux_ref):
    _attn_kernel(x_ref, ln1_ref, wq_ref, wk_ref, wv_ref, wo_ref, qn_ref,
                 kn_ref, cos_ref, sin_ref, bd_ref, pm_ref, xa_ref)
    xa = xa_ref[...]
    h2 = _rms(xa, ln2_ref[...])
    logits = jnp.dot(h2, wr_ref[...], preferred_element_type=jnp.float32)
    m = jnp.max(logits, axis=1, keepdims=True)
    ex = jnp.exp(logits - m)
    probs = ex * (1.0 / jnp.sum(ex, axis=1, keepdims=True))
    lane = jax.lax.broadcasted_iota(jnp.int32, (T, E), 1)
    v1 = jnp.max(probs, axis=1, keepdims=True)
    i1 = jnp.min(jnp.where(probs == v1, lane, E), axis=1, keepdims=True)
    oh1 = lane == i1
    probs_m = jnp.where(oh1, -1.0, probs)
    v2 = jnp.max(probs_m, axis=1, keepdims=True)
    i2 = jnp.min(jnp.where(probs_m == v2, lane, E), axis=1, keepdims=True)
    oh2 = lane == i2
    sg = v1 + v2
    g1 = v1 / sg
    g2 = v2 / sg
    ohall = oh1.astype(jnp.float32) + oh2.astype(jnp.float32)
    f_ = jnp.sum(ohall, axis=0, keepdims=True) / (T * TOPK)
    p_ = jnp.sum(probs, axis=0, keepdims=True) / T
    aux_ref[...] = E * jnp.sum(f_ * p_, axis=1, keepdims=True)
    rT = jax.lax.broadcasted_iota(jnp.int32, (T, T), 0)
    cT = jax.lax.broadcasted_iota(jnp.int32, (T, T), 1)
    ltok = (cT < rT).astype(jnp.float32)
    rank = jax.lax.dot_general(ltok, ohall, (((1,), (0,)), ((), ())),
                               preferred_element_type=jnp.float32)
    rE = jax.lax.broadcasted_iota(jnp.int32, (E, E), 0)
    cE = jax.lax.broadcasted_iota(jnp.int32, (E, E), 1)
    lexp_row = (rE < cE).astype(jnp.float32)
    hist_row = jnp.sum(ohall, axis=0, keepdims=True)
    phist_row = jnp.ceil(hist_row * 0.125) * 8.0
    poff_row = jnp.dot(phist_row, lexp_row, preferred_element_type=jnp.float32)
    pos_te = poff_row + rank
    pos1 = jnp.sum(jnp.where(oh1, pos_te, 0.0), axis=1, keepdims=True)
    pos2 = jnp.sum(jnp.where(oh2, pos_te, 0.0), axis=1, keepdims=True)
    pcol = jax.lax.broadcasted_iota(jnp.int32, (T, P), 1)
    m1 = (pcol == pos1.astype(jnp.int32)).astype(jnp.float32)
    m2 = (pcol == pos2.astype(jnp.int32)).astype(jnp.float32)
    smat = (m1 + m2).astype(jnp.bfloat16)
    xs_ref[...] = jax.lax.dot_general(smat, h2.astype(jnp.bfloat16),
                                      (((0,), (0,)), ((), ())),
                                      preferred_element_type=jnp.float32)
    sg_ref[...] = m1 * g1 + m2 * g2
    offs_ref[...] = (poff_row * 0.125).astype(jnp.int32)
    cnts_ref[...] = hist_row.astype(jnp.int32)


def _moe_kernel(offs_ref, cnts_ref, x_ref, xa_ref, sg_ref,
                wg_ref, wu_ref, wd_ref, o_ref, y_ref):
    eb = pl.program_id(0)

    @pl.when(eb == 0)
    def _():
        y_ref[...] = jnp.zeros((P, D), jnp.float32)

    rows = jax.lax.broadcasted_iota(jnp.int32, (TM, D), 0)

    for sub in range(EB):
        e = eb * EB + sub
        off8 = offs_ref[0, e]
        cnt = cnts_ref[0, e]
        nt = (cnt + TM - 1) // TM

        def body(j, _, off8=off8, cnt=cnt, sub=sub):
            base = (off8 + j * (TM // 8)) * 8
            xt = x_ref[pl.ds(base, TM), :]
            g = jnp.dot(xt, wg_ref[sub], preferred_element_type=jnp.float32)
            u = jnp.dot(xt, wu_ref[sub], preferred_element_type=jnp.float32)
            hid = (g / (1.0 + jnp.exp(-g))) * u
            y = jnp.dot(hid, wd_ref[sub], preferred_element_type=jnp.float32)
            mask = rows < cnt - j * TM
            y_ref[pl.ds(base, TM), :] = jnp.where(mask, y,
                                                  y_ref[pl.ds(base, TM), :])
            return 0

        jax.lax.fori_loop(0, nt, body, 0)

    @pl.when(eb == E // EB - 1)
    def _():
        sgb = sg_ref[...].astype(jnp.bfloat16)
        yb = y_ref[...].astype(jnp.bfloat16)
        o_ref[...] = xa_ref[...] + jnp.dot(sgb, yb,
                                           preferred_element_type=jnp.float32)


def kernel(x, ln1_w, wq, wk, wv, wo, q_norm_w, k_norm_w, ln2_w, w_router,
           wg, wu, wd):
    x2 = x.reshape(T, D)
    cos, sin, bd, pm = _rope_const()
    xa, xs, sgm, offs, cnts, aux = pl.pallas_call(
        _attn_route_kernel,
        out_shape=(
            jax.ShapeDtypeStruct((T, D), jnp.float32),
            jax.ShapeDtypeStruct((P, D), jnp.float32),
            jax.ShapeDtypeStruct((T, P), jnp.float32),
            jax.ShapeDtypeStruct((1, E), jnp.int32),
            jax.ShapeDtypeStruct((1, E), jnp.int32),
            jax.ShapeDtypeStruct((1, 1), jnp.float32),
        ),
    )(x2, ln1_w.reshape(1, D), wq, wk, wv, wo,
      jnp.tile(q_norm_w.reshape(1, HD), (1, H)),
      jnp.tile(k_norm_w.reshape(1, HD), (1, KV)), cos, sin, bd, pm,
      ln2_w.reshape(1, D), w_router)

    grid_spec = pltpu.PrefetchScalarGridSpec(
        num_scalar_prefetch=2,
        grid=(E // EB,),
        in_specs=[
            pl.BlockSpec((P, D), lambda e, offs, cnts: (0, 0)),
            pl.BlockSpec((T, D), lambda e, offs, cnts: (0, 0)),
            pl.BlockSpec((T, P), lambda e, offs, cnts: (0, 0)),
            pl.BlockSpec((EB, D, FF), lambda e, offs, cnts: (e, 0, 0)),
            pl.BlockSpec((EB, D, FF), lambda e, offs, cnts: (e, 0, 0)),
            pl.BlockSpec((EB, FF, D), lambda e, offs, cnts: (e, 0, 0)),
        ],
        out_specs=pl.BlockSpec((T, D), lambda e, offs, cnts: (0, 0)),
        scratch_shapes=[pltpu.VMEM((P, D), jnp.float32)],
    )
    out = pl.pallas_call(
        _moe_kernel,
        grid_spec=grid_spec,
        out_shape=jax.ShapeDtypeStruct((T, D), jnp.float32),
    )(offs, cnts, xs, xa, sgm, wg, wu, wd)
    return out.reshape(B, S, D), aux[0, 0]
